# Initial kernel scaffold; baseline (speedup 1.0000x reference)
import jax
import jax.numpy as jnp
from jax.experimental import pallas as pl


def kernel(feat, edge_index, mask_candidate_set, W0, b0, Wg1, bg1, Wg2, bg2, Wg3, bg3, Ws1, bs1, Ws2, bs2, Wt1, bt1, Wt2, bt2):
    raise NotImplementedError("write your pallas kernel here")


# speedup vs baseline: 12.4344x; 12.4344x over previous
import functools

import jax
import jax.numpy as jnp
from jax import lax
from jax.experimental import pallas as pl
from jax.experimental.pallas import tpu as pltpu
from jax.experimental.pallas import tpu_sc as plsc

N = 10000
E = 160000
MAXN = 9993

NC = 2
NS = 16
NW = NC * NS
CHUNK = 128
NCHUNK = E // CHUNK
CHUNKS_PER_TILE = -(-NCHUNK // NW)
NPAD = 10240
ROWS_PER_TILE = NPAD // NS


def _make_edge_pass(F):
  mesh = plsc.VectorSubcoreMesh(core_axis_name="c", subcore_axis_name="s")

  @functools.partial(
      pl.kernel,
      mesh=mesh,
      out_type=jax.ShapeDtypeStruct((NC, NPAD, F), jnp.float32),
      scratch_types=[
          pltpu.VMEM((CHUNK,), jnp.int32),
          pltpu.VMEM((CHUNK,), jnp.int32),
          pltpu.VMEM((CHUNK, F), jnp.float32),
          pltpu.VMEM((ROWS_PER_TILE, F), jnp.float32),
          pltpu.VMEM_SHARED((NPAD, F), jnp.float32),
          pltpu.SemaphoreType.DMA,
      ],
      compiler_params=pltpu.CompilerParams(use_tc_tiling_on_sc=False),
  )
  def edge_pass(y_hbm, src_hbm, dst_hbm, zeros_hbm, out_hbm,
                sidx, didx, rows, stage, acc, sem):
    c = lax.axis_index("c")
    s = lax.axis_index("s")
    w = s * NC + c
    roff = s * ROWS_PER_TILE

    pltpu.sync_copy(zeros_hbm, stage)
    pltpu.sync_copy(stage, acc.at[pl.ds(roff, ROWS_PER_TILE)])
    plsc.subcore_barrier()

    def body(i, carry):
      cid = w + i * NW

      @pl.when(cid < NCHUNK)
      def _():
        eoff = cid * CHUNK
        pltpu.sync_copy(src_hbm.at[pl.ds(eoff, CHUNK)], sidx)
        pltpu.sync_copy(dst_hbm.at[pl.ds(eoff, CHUNK)], didx)
        pltpu.async_copy(y_hbm.at[sidx], rows, sem).wait()
        pltpu.sync_copy(rows, acc.at[didx], add=True)

      return carry

    lax.fori_loop(0, CHUNKS_PER_TILE, body, 0)
    plsc.subcore_barrier()

    pltpu.sync_copy(acc.at[pl.ds(roff, ROWS_PER_TILE)], stage)
    pltpu.sync_copy(stage, out_hbm.at[c, pl.ds(roff, ROWS_PER_TILE)])

  return edge_pass


def _relu6(x):
  return jnp.clip(x, 0.0, 6.0)


def _entry_body(feat_ref, w_ref, b_ref, out_ref):
  out_ref[...] = _relu6(
      jnp.dot(feat_ref[...], w_ref[...], preferred_element_type=jnp.float32)
      + b_ref[...])


def _deg_body(degp_ref, x0_ref, w_ref, dinv_ref, y_ref):
  deg = degp_ref[0, :N, 0:1] + degp_ref[1, :N, 0:1] + 1.0
  dinv = lax.rsqrt(jnp.maximum(deg, 1e-12))
  dinv_ref[...] = dinv
  y_ref[...] = jnp.dot(x0_ref[...], w_ref[...],
                       preferred_element_type=jnp.float32) * dinv


def _layer_body(zp_ref, y_ref, dinv_ref, b_ref, wn_ref, yn_ref):
  h = _relu6((zp_ref[0, :N] + zp_ref[1, :N] + y_ref[...]) * dinv_ref[...]
             + b_ref[...])
  yn_ref[...] = jnp.dot(h, wn_ref[...],
                        preferred_element_type=jnp.float32) * dinv_ref[...]


def _head_body(zp_ref, y_ref, dinv_ref, bg3_ref, ws1_ref, bs1_ref, ws2_ref,
               bs2_ref, wt1a_ref, wt1b_ref, bt1_ref, wt2_ref, bt2_ref,
               mask_ref, sprob_ref, sidx_ref, tprob_ref, tidx_ref):
  x = _relu6((zp_ref[0, :N] + zp_ref[1, :N] + y_ref[...]) * dinv_ref[...]
             + bg3_ref[...])
  sh = _relu6(jnp.dot(x, ws1_ref[...], preferred_element_type=jnp.float32)
              + bs1_ref[...])
  sl = jnp.dot(sh, ws2_ref[...], preferred_element_type=jnp.float32) \
      + bs2_ref[...]
  sp = jnp.exp(sl - jnp.max(sl))
  sp = sp / jnp.sum(sp)
  m = mask_ref[...] > 0.0
  sprob_ref[...] = jnp.where(m, 0.0, sp)
  rows = lax.broadcasted_iota(jnp.int32, (N, 1), 0)
  sm = jnp.where(m, -1.0, sp)
  smx = jnp.max(sm)
  sidx = jnp.min(jnp.where(sm == smx, rows, N))
  sidx_ref[...] = jnp.reshape(sidx, (1, 1))
  xs = jnp.sum(jnp.where(rows == sidx, x, 0.0), axis=0, keepdims=True)
  th = _relu6(jnp.dot(x, wt1a_ref[...], preferred_element_type=jnp.float32)
              + jnp.dot(xs, wt1b_ref[...], preferred_element_type=jnp.float32)
              + bt1_ref[...])
  tl = jnp.dot(th, wt2_ref[...], preferred_element_type=jnp.float32) \
      + bt2_ref[...]
  tp = jnp.exp(tl - jnp.max(tl))
  tp = tp / jnp.sum(tp)
  tmask = rows < MAXN
  tprob_ref[...] = jnp.where(tmask, tp, 0.0)
  tmx = jnp.max(jnp.where(tmask, tp, -1.0))
  tidx = jnp.min(jnp.where((tp == tmx) & tmask, rows, N))
  tidx_ref[...] = jnp.reshape(tidx, (1, 1))


def _tc_call(body, out_shapes):
  return pl.pallas_call(
      body,
      out_shape=out_shapes,
  )


def kernel(feat, edge_index, mask_candidate_set, W0, b0, Wg1, bg1, Wg2, bg2,
           Wg3, bg3, Ws1, bs1, Ws2, bs2, Wt1, bt1, Wt2, bt2):
  src = edge_index[0].astype(jnp.int32)
  dst = edge_index[1].astype(jnp.int32)
  f32 = jnp.float32

  x0 = _tc_call(_entry_body, jax.ShapeDtypeStruct((N, 8), f32))(
      feat, W0, b0.reshape(1, 8))

  ones8 = jnp.ones((N, 8), f32)
  degp = _make_edge_pass(8)(ones8, src, dst, jnp.zeros((ROWS_PER_TILE, 8), f32))

  dinv, y1 = _tc_call(
      _deg_body,
      (jax.ShapeDtypeStruct((N, 1), f32), jax.ShapeDtypeStruct((N, 16), f32)),
  )(degp, x0, Wg1)

  z1 = _make_edge_pass(16)(y1, src, dst, jnp.zeros((ROWS_PER_TILE, 16), f32))
  y2 = _tc_call(_layer_body, jax.ShapeDtypeStruct((N, 24), f32))(
      z1, y1, dinv, bg1.reshape(1, 16), Wg2)

  z2 = _make_edge_pass(24)(y2, src, dst, jnp.zeros((ROWS_PER_TILE, 24), f32))
  y3 = _tc_call(_layer_body, jax.ShapeDtypeStruct((N, 32), f32))(
      z2, y2, dinv, bg2.reshape(1, 24), Wg3)

  z3 = _make_edge_pass(32)(y3, src, dst, jnp.zeros((ROWS_PER_TILE, 32), f32))

  sprob, sidx, tprob, tidx = _tc_call(
      _head_body,
      (jax.ShapeDtypeStruct((N, 1), f32),
       jax.ShapeDtypeStruct((1, 1), jnp.int32),
       jax.ShapeDtypeStruct((N, 1), f32),
       jax.ShapeDtypeStruct((1, 1), jnp.int32)),
  )(z3, y3, dinv, bg3.reshape(1, 32), Ws1, bs1.reshape(1, 16), Ws2,
    bs2.reshape(1, 1), Wt1[:32], Wt1[32:], bt1.reshape(1, 24), Wt2,
    bt2.reshape(1, 1), mask_candidate_set.astype(f32).reshape(N, 1))

  return sprob, sidx[0, 0], tprob, tidx[0, 0]


# speedup vs baseline: 16.7444x; 1.3466x over previous
import functools

import jax
import jax.numpy as jnp
from jax import lax
from jax.experimental import pallas as pl
from jax.experimental.pallas import tpu as pltpu
from jax.experimental.pallas import tpu_sc as plsc

N = 10000
E = 160000
MAXN = 9993

NC = 2
NS = 16
NW = NC * NS
CHUNK = 128
NCHUNK = E // CHUNK
NPAD = 10240
ROWS_PER_TILE = NPAD // NS
GK = 8
CHUNKS_PER_TILE = 40
NGROUP = CHUNKS_PER_TILE // GK
EPAD = NW * CHUNKS_PER_TILE * CHUNK


def _make_edge_pass(F):
  mesh = plsc.VectorSubcoreMesh(core_axis_name="c", subcore_axis_name="s")

  @functools.partial(
      pl.kernel,
      mesh=mesh,
      out_type=jax.ShapeDtypeStruct((NC, NPAD, F), jnp.float32),
      scratch_types=[
          pltpu.VMEM((CHUNKS_PER_TILE, CHUNK), jnp.int32),
          pltpu.VMEM((CHUNKS_PER_TILE, CHUNK), jnp.int32),
          pltpu.VMEM((GK, CHUNK, F), jnp.float32),
          pltpu.VMEM((ROWS_PER_TILE, F), jnp.float32),
          pltpu.VMEM_SHARED((NPAD, F), jnp.float32),
          pltpu.SemaphoreType.DMA,
          pltpu.SemaphoreType.DMA,
      ],
      compiler_params=pltpu.CompilerParams(use_tc_tiling_on_sc=False),
  )
  def edge_pass(y_hbm, src_hbm, dst_hbm, zeros_hbm, out_hbm,
                sidx, didx, rows, stage, acc, semg, sems):
    c = lax.axis_index("c")
    s = lax.axis_index("s")
    w = s * NC + c
    roff = s * ROWS_PER_TILE

    pltpu.async_copy(src_hbm.at[pl.ds(w * CHUNKS_PER_TILE, CHUNKS_PER_TILE)],
                     sidx, sems)
    pltpu.async_copy(dst_hbm.at[pl.ds(w * CHUNKS_PER_TILE, CHUNKS_PER_TILE)],
                     didx, sems)
    pltpu.sync_copy(zeros_hbm, stage)
    pltpu.sync_copy(stage, acc.at[pl.ds(roff, ROWS_PER_TILE)])
    pltpu.make_async_copy(src_hbm.at[pl.ds(0, CHUNKS_PER_TILE)], sidx,
                          sems).wait()
    pltpu.make_async_copy(src_hbm.at[pl.ds(0, CHUNKS_PER_TILE)], didx,
                          sems).wait()
    plsc.subcore_barrier()

    def group(g, carry):
      gds = []
      for j in range(GK):
        i = g * GK + j
        gds.append(pltpu.async_copy(y_hbm.at[sidx.at[i]], rows.at[j], semg))
      for d in gds:
        d.wait()
      sds = []
      for j in range(GK):
        i = g * GK + j
        sds.append(pltpu.async_copy(rows.at[j], acc.at[didx.at[i]], sems,
                                    add=True))
      for d in sds:
        d.wait()
      return carry

    lax.fori_loop(0, NGROUP, group, 0)
    plsc.subcore_barrier()

    pltpu.sync_copy(acc.at[pl.ds(roff, ROWS_PER_TILE)], stage)
    pltpu.sync_copy(stage, out_hbm.at[c, pl.ds(roff, ROWS_PER_TILE)])

  return edge_pass


def _relu6(x):
  return jnp.clip(x, 0.0, 6.0)


def _entry_body(feat_ref, w_ref, b_ref, out_ref):
  out_ref[...] = _relu6(
      jnp.dot(feat_ref[...], w_ref[...], preferred_element_type=jnp.float32)
      + b_ref[...])


def _deg_body(degp_ref, x0_ref, w_ref, dinv_ref, y_ref):
  deg = degp_ref[0, :N, 0:1] + degp_ref[1, :N, 0:1] + 1.0
  dinv = lax.rsqrt(jnp.maximum(deg, 1e-12))
  dinv_ref[...] = dinv
  y_ref[...] = jnp.dot(x0_ref[...], w_ref[...],
                       preferred_element_type=jnp.float32) * dinv


def _layer_body(zp_ref, y_ref, dinv_ref, b_ref, wn_ref, yn_ref):
  h = _relu6((zp_ref[0, :N] + zp_ref[1, :N] + y_ref[...]) * dinv_ref[...]
             + b_ref[...])
  yn_ref[...] = jnp.dot(h, wn_ref[...],
                        preferred_element_type=jnp.float32) * dinv_ref[...]


def _head_body(zp_ref, y_ref, dinv_ref, bg3_ref, ws1_ref, bs1_ref, ws2_ref,
               bs2_ref, wt1a_ref, wt1b_ref, bt1_ref, wt2_ref, bt2_ref,
               mask_ref, sprob_ref, sidx_ref, tprob_ref, tidx_ref):
  x = _relu6((zp_ref[0, :N] + zp_ref[1, :N] + y_ref[...]) * dinv_ref[...]
             + bg3_ref[...])
  sh = _relu6(jnp.dot(x, ws1_ref[...], preferred_element_type=jnp.float32)
              + bs1_ref[...])
  sl = jnp.dot(sh, ws2_ref[...], preferred_element_type=jnp.float32) \
      + bs2_ref[...]
  sp = jnp.exp(sl - jnp.max(sl))
  sp = sp / jnp.sum(sp)
  m = mask_ref[...] > 0.0
  sprob_ref[...] = jnp.where(m, 0.0, sp)
  rows = lax.broadcasted_iota(jnp.int32, (N, 1), 0)
  sm = jnp.where(m, -1.0, sp)
  smx = jnp.max(sm)
  sidx = jnp.min(jnp.where(sm == smx, rows, N))
  sidx_ref[...] = jnp.reshape(sidx, (1, 1))
  xs = jnp.sum(jnp.where(rows == sidx, x, 0.0), axis=0, keepdims=True)
  th = _relu6(jnp.dot(x, wt1a_ref[...], preferred_element_type=jnp.float32)
              + jnp.dot(xs, wt1b_ref[...], preferred_element_type=jnp.float32)
              + bt1_ref[...])
  tl = jnp.dot(th, wt2_ref[...], preferred_element_type=jnp.float32) \
      + bt2_ref[...]
  tp = jnp.exp(tl - jnp.max(tl))
  tp = tp / jnp.sum(tp)
  tmask = rows < MAXN
  tprob_ref[...] = jnp.where(tmask, tp, 0.0)
  tmx = jnp.max(jnp.where(tmask, tp, -1.0))
  tidx = jnp.min(jnp.where((tp == tmx) & tmask, rows, N))
  tidx_ref[...] = jnp.reshape(tidx, (1, 1))


def _tc_call(body, out_shapes):
  return pl.pallas_call(
      body,
      out_shape=out_shapes,
  )


def kernel(feat, edge_index, mask_candidate_set, W0, b0, Wg1, bg1, Wg2, bg2,
           Wg3, bg3, Ws1, bs1, Ws2, bs2, Wt1, bt1, Wt2, bt2):
  f32 = jnp.float32
  src = jnp.concatenate(
      [edge_index[0].astype(jnp.int32),
       jnp.zeros((EPAD - E,), jnp.int32)]).reshape(EPAD // CHUNK, CHUNK)
  dst = jnp.concatenate(
      [edge_index[1].astype(jnp.int32),
       jnp.full((EPAD - E,), N, jnp.int32)]).reshape(EPAD // CHUNK, CHUNK)

  x0 = _tc_call(_entry_body, jax.ShapeDtypeStruct((N, 8), f32))(
      feat, W0, b0.reshape(1, 8))

  ones8 = jnp.ones((N, 8), f32)
  degp = _make_edge_pass(8)(ones8, src, dst, jnp.zeros((ROWS_PER_TILE, 8), f32))

  dinv, y1 = _tc_call(
      _deg_body,
      (jax.ShapeDtypeStruct((N, 1), f32), jax.ShapeDtypeStruct((N, 16), f32)),
  )(degp, x0, Wg1)

  z1 = _make_edge_pass(16)(y1, src, dst, jnp.zeros((ROWS_PER_TILE, 16), f32))
  y2 = _tc_call(_layer_body, jax.ShapeDtypeStruct((N, 24), f32))(
      z1, y1, dinv, bg1.reshape(1, 16), Wg2)

  z2 = _make_edge_pass(24)(y2, src, dst, jnp.zeros((ROWS_PER_TILE, 24), f32))
  y3 = _tc_call(_layer_body, jax.ShapeDtypeStruct((N, 32), f32))(
      z2, y2, dinv, bg2.reshape(1, 24), Wg3)

  z3 = _make_edge_pass(32)(y3, src, dst, jnp.zeros((ROWS_PER_TILE, 32), f32))

  sprob, sidx, tprob, tidx = _tc_call(
      _head_body,
      (jax.ShapeDtypeStruct((N, 1), f32),
       jax.ShapeDtypeStruct((1, 1), jnp.int32),
       jax.ShapeDtypeStruct((N, 1), f32),
       jax.ShapeDtypeStruct((1, 1), jnp.int32)),
  )(z3, y3, dinv, bg3.reshape(1, 32), Ws1, bs1.reshape(1, 16), Ws2,
    bs2.reshape(1, 1), Wt1[:32], Wt1[32:], bt1.reshape(1, 24), Wt2,
    bt2.reshape(1, 1), mask_candidate_set.astype(f32).reshape(N, 1))

  return sprob, sidx[0, 0], tprob, tidx[0, 0]


# speedup vs baseline: 17.4232x; 1.0405x over previous
import functools

import jax
import jax.numpy as jnp
from jax import lax
from jax.experimental import pallas as pl
from jax.experimental.pallas import tpu as pltpu
from jax.experimental.pallas import tpu_sc as plsc

N = 10000
E = 160000
MAXN = 9993

NC = 2
NS = 16
NW = NC * NS
CHUNK = 128
NCHUNK = E // CHUNK
NPAD = 10240
ROWS_PER_TILE = NPAD // NS
GK = 8
CHUNKS_PER_TILE = 40
NGROUP = CHUNKS_PER_TILE // GK
EPAD = NW * CHUNKS_PER_TILE * CHUNK


def _make_edge_pass(F):
  mesh = plsc.VectorSubcoreMesh(core_axis_name="c", subcore_axis_name="s")

  @functools.partial(
      pl.kernel,
      mesh=mesh,
      out_type=jax.ShapeDtypeStruct((NC, NPAD, F), jnp.float32),
      scratch_types=[
          pltpu.VMEM((CHUNKS_PER_TILE, CHUNK), jnp.int32),
          pltpu.VMEM((CHUNKS_PER_TILE, CHUNK), jnp.int32),
          pltpu.VMEM((2, GK, CHUNK, F), jnp.float32),
          pltpu.VMEM((ROWS_PER_TILE, F), jnp.float32),
          pltpu.VMEM_SHARED((NPAD, F), jnp.float32),
          pltpu.SemaphoreType.DMA,
          pltpu.SemaphoreType.DMA,
      ],
      compiler_params=pltpu.CompilerParams(use_tc_tiling_on_sc=False),
  )
  def edge_pass(y_hbm, src_hbm, dst_hbm, zeros_hbm, out_hbm,
                sidx, didx, rows, stage, acc, semg, sems):
    c = lax.axis_index("c")
    s = lax.axis_index("s")
    w = s * NC + c
    roff = s * ROWS_PER_TILE

    pltpu.async_copy(src_hbm.at[pl.ds(w * CHUNKS_PER_TILE, CHUNKS_PER_TILE)],
                     sidx, sems)
    pltpu.async_copy(dst_hbm.at[pl.ds(w * CHUNKS_PER_TILE, CHUNKS_PER_TILE)],
                     didx, sems)
    pltpu.sync_copy(zeros_hbm, stage)
    pltpu.sync_copy(stage, acc.at[pl.ds(roff, ROWS_PER_TILE)])
    pltpu.make_async_copy(src_hbm.at[pl.ds(0, CHUNKS_PER_TILE)], sidx,
                          sems).wait()
    pltpu.make_async_copy(src_hbm.at[pl.ds(0, CHUNKS_PER_TILE)], didx,
                          sems).wait()
    plsc.subcore_barrier()

    def fire_gathers(g, bank):
      for j in range(GK):
        pltpu.async_copy(y_hbm.at[sidx.at[g * GK + j]], rows.at[bank, j],
                         semg)

    def drain_gathers(bank):
      for j in range(GK):
        pltpu.make_async_copy(y_hbm.at[sidx.at[0]], rows.at[bank, j],
                              semg).wait()

    def drain_scatters(bank):
      for j in range(GK):
        pltpu.make_async_copy(rows.at[bank, j], acc.at[didx.at[0]],
                              sems).wait()

    fire_gathers(0, 0)

    def group(g, carry):
      bank = lax.rem(g, 2)
      drain_gathers(bank)

      @pl.when(g + 1 < NGROUP)
      def _():
        fire_gathers(g + 1, 1 - bank)

      for j in range(GK):
        pltpu.async_copy(rows.at[bank, j], acc.at[didx.at[g * GK + j]], sems,
                         add=True)
      drain_scatters(bank)
      return carry

    lax.fori_loop(0, NGROUP, group, 0)
    plsc.subcore_barrier()

    pltpu.sync_copy(acc.at[pl.ds(roff, ROWS_PER_TILE)], stage)
    pltpu.sync_copy(stage, out_hbm.at[c, pl.ds(roff, ROWS_PER_TILE)])

  return edge_pass


def _relu6(x):
  return jnp.clip(x, 0.0, 6.0)


def _entry_body(feat_ref, w_ref, b_ref, out_ref):
  out_ref[...] = _relu6(
      jnp.dot(feat_ref[...], w_ref[...], preferred_element_type=jnp.float32)
      + b_ref[...])


def _deg_body(degp_ref, x0_ref, w_ref, dinv_ref, y_ref):
  deg = degp_ref[0, :N, 0:1] + degp_ref[1, :N, 0:1] + 1.0
  dinv = lax.rsqrt(jnp.maximum(deg, 1e-12))
  dinv_ref[...] = dinv
  y_ref[...] = jnp.dot(x0_ref[...], w_ref[...],
                       preferred_element_type=jnp.float32) * dinv


def _layer_body(zp_ref, y_ref, dinv_ref, b_ref, wn_ref, yn_ref):
  h = _relu6((zp_ref[0, :N] + zp_ref[1, :N] + y_ref[...]) * dinv_ref[...]
             + b_ref[...])
  yn_ref[...] = jnp.dot(h, wn_ref[...],
                        preferred_element_type=jnp.float32) * dinv_ref[...]


def _head_body(zp_ref, y_ref, dinv_ref, bg3_ref, ws1_ref, bs1_ref, ws2_ref,
               bs2_ref, wt1a_ref, wt1b_ref, bt1_ref, wt2_ref, bt2_ref,
               mask_ref, sprob_ref, sidx_ref, tprob_ref, tidx_ref):
  x = _relu6((zp_ref[0, :N] + zp_ref[1, :N] + y_ref[...]) * dinv_ref[...]
             + bg3_ref[...])
  sh = _relu6(jnp.dot(x, ws1_ref[...], preferred_element_type=jnp.float32)
              + bs1_ref[...])
  sl = jnp.dot(sh, ws2_ref[...], preferred_element_type=jnp.float32) \
      + bs2_ref[...]
  sp = jnp.exp(sl - jnp.max(sl))
  sp = sp / jnp.sum(sp)
  m = mask_ref[...] > 0.0
  sprob_ref[...] = jnp.where(m, 0.0, sp)
  rows = lax.broadcasted_iota(jnp.int32, (N, 1), 0)
  sm = jnp.where(m, -1.0, sp)
  smx = jnp.max(sm)
  sidx = jnp.min(jnp.where(sm == smx, rows, N))
  sidx_ref[...] = jnp.reshape(sidx, (1, 1))
  xs = jnp.sum(jnp.where(rows == sidx, x, 0.0), axis=0, keepdims=True)
  th = _relu6(jnp.dot(x, wt1a_ref[...], preferred_element_type=jnp.float32)
              + jnp.dot(xs, wt1b_ref[...], preferred_element_type=jnp.float32)
              + bt1_ref[...])
  tl = jnp.dot(th, wt2_ref[...], preferred_element_type=jnp.float32) \
      + bt2_ref[...]
  tp = jnp.exp(tl - jnp.max(tl))
  tp = tp / jnp.sum(tp)
  tmask = rows < MAXN
  tprob_ref[...] = jnp.where(tmask, tp, 0.0)
  tmx = jnp.max(jnp.where(tmask, tp, -1.0))
  tidx = jnp.min(jnp.where((tp == tmx) & tmask, rows, N))
  tidx_ref[...] = jnp.reshape(tidx, (1, 1))


def _tc_call(body, out_shapes):
  return pl.pallas_call(
      body,
      out_shape=out_shapes,
  )


def kernel(feat, edge_index, mask_candidate_set, W0, b0, Wg1, bg1, Wg2, bg2,
           Wg3, bg3, Ws1, bs1, Ws2, bs2, Wt1, bt1, Wt2, bt2):
  f32 = jnp.float32
  src = jnp.concatenate(
      [edge_index[0].astype(jnp.int32),
       jnp.zeros((EPAD - E,), jnp.int32)]).reshape(EPAD // CHUNK, CHUNK)
  dst = jnp.concatenate(
      [edge_index[1].astype(jnp.int32),
       jnp.full((EPAD - E,), N, jnp.int32)]).reshape(EPAD // CHUNK, CHUNK)

  x0 = _tc_call(_entry_body, jax.ShapeDtypeStruct((N, 8), f32))(
      feat, W0, b0.reshape(1, 8))

  ones8 = jnp.ones((N, 8), f32)
  degp = _make_edge_pass(8)(ones8, src, dst, jnp.zeros((ROWS_PER_TILE, 8), f32))

  dinv, y1 = _tc_call(
      _deg_body,
      (jax.ShapeDtypeStruct((N, 1), f32), jax.ShapeDtypeStruct((N, 16), f32)),
  )(degp, x0, Wg1)

  z1 = _make_edge_pass(16)(y1, src, dst, jnp.zeros((ROWS_PER_TILE, 16), f32))
  y2 = _tc_call(_layer_body, jax.ShapeDtypeStruct((N, 24), f32))(
      z1, y1, dinv, bg1.reshape(1, 16), Wg2)

  z2 = _make_edge_pass(24)(y2, src, dst, jnp.zeros((ROWS_PER_TILE, 24), f32))
  y3 = _tc_call(_layer_body, jax.ShapeDtypeStruct((N, 32), f32))(
      z2, y2, dinv, bg2.reshape(1, 24), Wg3)

  z3 = _make_edge_pass(32)(y3, src, dst, jnp.zeros((ROWS_PER_TILE, 32), f32))

  sprob, sidx, tprob, tidx = _tc_call(
      _head_body,
      (jax.ShapeDtypeStruct((N, 1), f32),
       jax.ShapeDtypeStruct((1, 1), jnp.int32),
       jax.ShapeDtypeStruct((N, 1), f32),
       jax.ShapeDtypeStruct((1, 1), jnp.int32)),
  )(z3, y3, dinv, bg3.reshape(1, 32), Ws1, bs1.reshape(1, 16), Ws2,
    bs2.reshape(1, 1), Wt1[:32], Wt1[32:], bt1.reshape(1, 24), Wt2,
    bt2.reshape(1, 1), mask_candidate_set.astype(f32).reshape(N, 1))

  return sprob, sidx[0, 0], tprob, tidx[0, 0]
